# baseline (speedup 1.0000x reference)
import functools

import jax
import jax.numpy as jnp
from jax import lax
from jax.experimental import pallas as pl
from jax.experimental.pallas import tpu as pltpu
from jax.experimental.pallas import tpu_sc as plsc

N = 10000
E = 320000
C = 128
K = 6
EPS = 1e-5

NC = 2
NS = 16
NW = NC * NS
EPW = E // NW
CH = 80
NCHUNK = EPW // CH
NBUF = 4
NGRP = 32
NPAD = 10240
ROWS_PER = NPAD // NS

_mesh = plsc.VectorSubcoreMesh(core_axis_name="c", subcore_axis_name="s")

CH_D = 80
NBUF_D = 5
NGRP_D = EPW // (CH_D * NBUF_D)


def _fill(ref, rows, value):
    def body(r, carry):
        for j in range(C // 16):
            ref[r, pl.ds(j * 16, 16)] = jnp.full((16,), value, jnp.float32)
        return carry
    lax.fori_loop(0, rows, body, 0)


def _zero_accum_slice(staging, nrows, accum, r0):
    _fill(staging, nrows, 0.0)
    for j in range(ROWS_PER // nrows):
        pltpu.sync_copy(staging, accum.at[pl.ds(r0 + j * nrows, nrows)])


@functools.partial(
    pl.kernel,
    out_type=jax.ShapeDtypeStruct((NC, NPAD, C), jnp.float32),
    mesh=_mesh,
    scratch_types=[
        pltpu.VMEM((2, NBUF_D, CH_D), jnp.int32),
        pltpu.VMEM((CH_D, C), jnp.float32),
        pltpu.SemaphoreType.DMA,
        pltpu.SemaphoreType.DMA,
        pltpu.VMEM_SHARED((NPAD, C), jnp.float32),
    ],
)
def _sc_degree(row_hbm, out_hbm, rowg, ones_v, ssem, isem, accum):
    cid = lax.axis_index("c")
    sid = lax.axis_index("s")
    wid = sid * NC + cid
    r0 = sid * ROWS_PER
    _zero_accum_slice(ones_v, CH_D, accum, r0)
    _fill(ones_v, CH_D, 1.0)
    pltpu.sync_copy(row_hbm.at[wid, 0], rowg.at[0])
    plsc.subcore_barrier()

    def body(g, carry):
        p = lax.rem(g, 2)
        pn = lax.rem(g + 1, 2)

        @pl.when(g > 0)
        def _():
            for b in range(NBUF_D):
                pltpu.make_async_copy(ones_v, accum.at[rowg.at[p, 0]],
                                      ssem).wait()
            pltpu.make_async_copy(row_hbm.at[wid, 0], rowg.at[p],
                                  isem).wait()

        @pl.when(g + 1 < NGRP_D)
        def _():
            pltpu.async_copy(row_hbm.at[wid, g + 1], rowg.at[pn], isem)

        for b in range(NBUF_D):
            pltpu.async_copy(ones_v, accum.at[rowg.at[p, b]], ssem, add=True)

        return carry

    lax.fori_loop(0, NGRP_D, body, 0)
    for b in range(NBUF_D):
        pltpu.make_async_copy(ones_v, accum.at[rowg.at[0, 0]], ssem).wait()
    plsc.subcore_barrier()
    pltpu.sync_copy(accum.at[pl.ds(r0, ROWS_PER)],
                    out_hbm.at[cid, pl.ds(r0, ROWS_PER)])


@functools.partial(
    pl.kernel,
    out_type=jax.ShapeDtypeStruct((NC, NPAD, C), jnp.float32),
    mesh=_mesh,
    scratch_types=(
        [pltpu.VMEM((2, NBUF, CH), jnp.int32),
         pltpu.VMEM((2, NBUF, CH), jnp.int32)]
        + [pltpu.VMEM((CH, C), jnp.float32) for _ in range(NBUF)]
        + [pltpu.SemaphoreType.DMA for _ in range(2 * NBUF)]
        + [pltpu.SemaphoreType.DMA, pltpu.VMEM_SHARED((NPAD, C), jnp.float32)]
    ),
)
def _sc_segsum(g_hbm, col_hbm, row_hbm, out_hbm, colg, rowg, *rest):
    bufs = rest[:NBUF]
    gsems = rest[NBUF:2 * NBUF]
    ssems = rest[2 * NBUF:3 * NBUF]
    isem = rest[3 * NBUF]
    accum = rest[3 * NBUF + 1]
    cid = lax.axis_index("c")
    sid = lax.axis_index("s")
    wid = sid * NC + cid
    r0 = sid * ROWS_PER
    _zero_accum_slice(bufs[0], CH, accum, r0)
    pltpu.sync_copy(col_hbm.at[wid, 0], colg.at[0])
    pltpu.sync_copy(row_hbm.at[wid, 0], rowg.at[0])
    pltpu.async_copy(col_hbm.at[wid, 1], colg.at[1], isem)
    pltpu.async_copy(row_hbm.at[wid, 1], rowg.at[1], isem)
    plsc.subcore_barrier()

    for b in range(NBUF):
        pltpu.async_copy(g_hbm.at[colg.at[0, b]], bufs[b], gsems[b])

    def body(g, carry):
        p = lax.rem(g, 2)
        pn = lax.rem(g + 1, 2)

        pltpu.make_async_copy(col_hbm.at[wid, 0], colg.at[pn], isem).wait()
        pltpu.make_async_copy(row_hbm.at[wid, 0], rowg.at[pn], isem).wait()

        for b in range(NBUF):
            pltpu.make_async_copy(g_hbm.at[colg.at[p, b]], bufs[b],
                                  gsems[b]).wait()
            pltpu.sync_copy(bufs[b], accum.at[rowg.at[p, b]], add=True)
            pltpu.async_copy(g_hbm.at[colg.at[pn, b]], bufs[b], gsems[b])

        @pl.when(g + 2 < NGRP)
        def _():
            pltpu.async_copy(col_hbm.at[wid, g + 2], colg.at[p], isem)
            pltpu.async_copy(row_hbm.at[wid, g + 2], rowg.at[p], isem)

        return carry

    lax.fori_loop(0, NGRP - 1, body, 0)

    pl_ = lax.rem(NGRP - 1, 2)
    for b in range(NBUF):
        pltpu.make_async_copy(g_hbm.at[colg.at[pl_, b]], bufs[b],
                              gsems[b]).wait()
        pltpu.sync_copy(bufs[b], accum.at[rowg.at[pl_, b]], add=True)

    plsc.subcore_barrier()
    pltpu.sync_copy(accum.at[pl.ds(r0, ROWS_PER)],
                    out_hbm.at[cid, pl.ds(r0, ROWS_PER)])


_B = 2000
_GRID = N // _B


def _tc_prep_body(x_ref, d16_ref, dinvb_ref, d2b_ref, g0_ref):
    deg = d16_ref[0, :, 0:1] + d16_ref[1, :, 0:1]
    dinv = jnp.where(deg > 0.0, lax.rsqrt(jnp.maximum(deg, 1e-12)), 0.0)
    dinvb = jnp.broadcast_to(dinv, (_B, C))
    dinvb_ref[...] = dinvb
    d2b_ref[...] = dinvb * dinvb
    g0_ref[...] = dinvb * x_ref[...]


def _tc_combine_body(first, yp_ref, scale_ref, prev_ref, out_ref):
    y = yp_ref[0] + yp_ref[1]
    if first:
        out_ref[...] = -(scale_ref[...] * y)
    else:
        out_ref[...] = -2.0 * (scale_ref[...] * y) - prev_ref[...]


def _tc_final_body(yp_ref, dinvb_ref, tx0_ref, tx1_ref, tx2_ref, tx3_ref,
                   tx4_ref, w_ref, b_ref, o_ref):
    tx5 = -2.0 * (dinvb_ref[...] * (yp_ref[0] + yp_ref[1])) - tx3_ref[...]
    txs = (tx0_ref[...], tx1_ref[...], tx2_ref[...], tx3_ref[...],
           tx4_ref[...], tx5)
    h = b_ref[...]
    for k in range(K):
        h = h + jnp.dot(txs[k], w_ref[k], preferred_element_type=jnp.float32)
    m = jnp.mean(h, axis=1, keepdims=True)
    cen = h - m
    v = jnp.mean(cen * cen, axis=1, keepdims=True)
    o_ref[...] = jnp.maximum(cen * lax.rsqrt(v + EPS), 0.0)


_row_spec = pl.BlockSpec((_B, C), lambda i: (i, 0))
_yp_spec = pl.BlockSpec((NC, _B, C), lambda i: (0, i, 0))

_tc_prep = pl.pallas_call(
    _tc_prep_body,
    grid=(_GRID,),
    in_specs=[_row_spec, _yp_spec],
    out_specs=[_row_spec, _row_spec, _row_spec],
    out_shape=[jax.ShapeDtypeStruct((N, C), jnp.float32)] * 3,
)

_BC = 5000
_rowc_spec = pl.BlockSpec((_BC, C), lambda i: (i, 0))
_ypc_spec = pl.BlockSpec((NC, _BC, C), lambda i: (0, i, 0))

_tc_combine_first = pl.pallas_call(
    functools.partial(_tc_combine_body, True),
    grid=(N // _BC,),
    in_specs=[_ypc_spec, _rowc_spec, _rowc_spec],
    out_specs=_rowc_spec,
    out_shape=jax.ShapeDtypeStruct((N, C), jnp.float32),
)

_tc_combine_rest = pl.pallas_call(
    functools.partial(_tc_combine_body, False),
    grid=(N // _BC,),
    in_specs=[_ypc_spec, _rowc_spec, _rowc_spec],
    out_specs=_rowc_spec,
    out_shape=jax.ShapeDtypeStruct((N, C), jnp.float32),
)

_tc_final = pl.pallas_call(
    _tc_final_body,
    grid=(_GRID,),
    in_specs=[_yp_spec] + [_row_spec] * K
    + [pl.BlockSpec((K, C, C), lambda i: (0, 0, 0)),
       pl.BlockSpec((1, C), lambda i: (0, 0))],
    out_specs=_row_spec,
    out_shape=jax.ShapeDtypeStruct((N, C), jnp.float32),
)


def kernel(x, adj_indices, W, b):
    pad = NGRP * NBUF * CH - EPW
    pad_rows = jnp.broadcast_to(N + (jnp.arange(pad) % (NPAD - N)), (NW, pad))
    pad_cols = jnp.broadcast_to((jnp.arange(pad) * 97) % N, (NW, pad))
    row = jnp.concatenate(
        [adj_indices[0].reshape(NW, EPW), pad_rows.astype(jnp.int32)], axis=1)
    row = row.reshape(NW, NGRP, NBUF, CH)
    col = jnp.concatenate(
        [adj_indices[1].reshape(NW, EPW), pad_cols.astype(jnp.int32)], axis=1)
    col = col.reshape(NW, NGRP, NBUF, CH)
    row_d = adj_indices[0].reshape(NW, NGRP_D, NBUF_D, CH_D)

    d16 = _sc_degree(row_d)
    dinvb, d2b, g = _tc_prep(x, d16)

    gs = [g]
    txs = [x]
    for k in range(1, K - 1):
        yp = _sc_segsum(g, col, row)
        if k == 1:
            g = _tc_combine_first(yp, d2b, d2b)
            tx = _tc_combine_first(yp, dinvb, dinvb)
        else:
            g = _tc_combine_rest(yp, d2b, gs[k - 2])
            tx = _tc_combine_rest(yp, dinvb, txs[k - 2])
        gs.append(g)
        txs.append(tx)

    yp = _sc_segsum(g, col, row)
    return _tc_final(yp, dinvb, *txs, W, b.reshape(1, C))


# speedup vs baseline: 1.0106x; 1.0106x over previous
import functools

import jax
import jax.numpy as jnp
from jax import lax
from jax.experimental import pallas as pl
from jax.experimental.pallas import tpu as pltpu
from jax.experimental.pallas import tpu_sc as plsc

N = 10000
E = 320000
C = 128
K = 6
EPS = 1e-5

NC = 2
NS = 16
NW = NC * NS
EPW = E // NW
CH = 80
NCHUNK = EPW // CH
NBUF = 4
NGRP = 32
NPAD = 10240
ROWS_PER = NPAD // NS

_mesh = plsc.VectorSubcoreMesh(core_axis_name="c", subcore_axis_name="s")

CH_D = 80
NBUF_D = 5
NGRP_D = EPW // (CH_D * NBUF_D)


def _fill(ref, rows, value):
    def body(r, carry):
        for j in range(C // 16):
            ref[r, pl.ds(j * 16, 16)] = jnp.full((16,), value, jnp.float32)
        return carry
    lax.fori_loop(0, rows, body, 0)


def _zero_accum_slice(staging, nrows, accum, r0):
    _fill(staging, nrows, 0.0)
    for j in range(ROWS_PER // nrows):
        pltpu.sync_copy(staging, accum.at[pl.ds(r0 + j * nrows, nrows)])


@functools.partial(
    pl.kernel,
    out_type=jax.ShapeDtypeStruct((NC, NPAD, C), jnp.float32),
    mesh=_mesh,
    scratch_types=[
        pltpu.VMEM((2, NBUF_D, CH_D), jnp.int32),
        pltpu.VMEM((CH_D, C), jnp.float32),
        pltpu.SemaphoreType.DMA,
        pltpu.SemaphoreType.DMA,
        pltpu.VMEM_SHARED((NPAD, C), jnp.float32),
    ],
)
def _sc_degree(row_hbm, out_hbm, rowg, ones_v, ssem, isem, accum):
    cid = lax.axis_index("c")
    sid = lax.axis_index("s")
    wid = sid * NC + cid
    r0 = sid * ROWS_PER
    _zero_accum_slice(ones_v, CH_D, accum, r0)
    _fill(ones_v, CH_D, 1.0)
    pltpu.sync_copy(row_hbm.at[wid, 0], rowg.at[0])
    plsc.subcore_barrier()

    def body(g, carry):
        p = lax.rem(g, 2)
        pn = lax.rem(g + 1, 2)

        @pl.when(g > 0)
        def _():
            for b in range(NBUF_D):
                pltpu.make_async_copy(ones_v, accum.at[rowg.at[p, 0]],
                                      ssem).wait()
            pltpu.make_async_copy(row_hbm.at[wid, 0], rowg.at[p],
                                  isem).wait()

        @pl.when(g + 1 < NGRP_D)
        def _():
            pltpu.async_copy(row_hbm.at[wid, g + 1], rowg.at[pn], isem)

        for b in range(NBUF_D):
            pltpu.async_copy(ones_v, accum.at[rowg.at[p, b]], ssem, add=True)

        return carry

    lax.fori_loop(0, NGRP_D, body, 0)
    for b in range(NBUF_D):
        pltpu.make_async_copy(ones_v, accum.at[rowg.at[0, 0]], ssem).wait()
    plsc.subcore_barrier()
    pltpu.sync_copy(accum.at[pl.ds(r0, ROWS_PER)],
                    out_hbm.at[cid, pl.ds(r0, ROWS_PER)])


@functools.partial(
    pl.kernel,
    out_type=jax.ShapeDtypeStruct((NC, NPAD, C), jnp.float32),
    mesh=_mesh,
    scratch_types=(
        [pltpu.VMEM((2, NBUF, CH), jnp.int32),
         pltpu.VMEM((2, NBUF, CH), jnp.int32)]
        + [pltpu.VMEM((CH, C), jnp.float32) for _ in range(NBUF)]
        + [pltpu.SemaphoreType.DMA for _ in range(2 * NBUF)]
        + [pltpu.SemaphoreType.DMA, pltpu.VMEM_SHARED((NPAD, C), jnp.float32)]
    ),
)
def _sc_segsum(g_hbm, col_hbm, row_hbm, out_hbm, colg, rowg, *rest):
    bufs = rest[:NBUF]
    gsems = rest[NBUF:2 * NBUF]
    ssems = rest[2 * NBUF:3 * NBUF]
    isem = rest[3 * NBUF]
    accum = rest[3 * NBUF + 1]
    cid = lax.axis_index("c")
    sid = lax.axis_index("s")
    wid = sid * NC + cid
    r0 = sid * ROWS_PER
    _zero_accum_slice(bufs[0], CH, accum, r0)
    pltpu.sync_copy(col_hbm.at[wid, 0], colg.at[0])
    pltpu.sync_copy(row_hbm.at[wid, 0], rowg.at[0])
    pltpu.async_copy(col_hbm.at[wid, 1], colg.at[1], isem)
    pltpu.async_copy(row_hbm.at[wid, 1], rowg.at[1], isem)
    plsc.subcore_barrier()

    for b in range(NBUF):
        pltpu.async_copy(g_hbm.at[colg.at[0, b]], bufs[b], gsems[b])

    def body(g, carry):
        p = lax.rem(g, 2)
        pn = lax.rem(g + 1, 2)

        pltpu.make_async_copy(col_hbm.at[wid, 0], colg.at[pn], isem).wait()
        pltpu.make_async_copy(row_hbm.at[wid, 0], rowg.at[pn], isem).wait()

        for b in range(NBUF):
            pltpu.make_async_copy(g_hbm.at[colg.at[p, b]], bufs[b],
                                  gsems[b]).wait()
            pltpu.sync_copy(bufs[b], accum.at[rowg.at[p, b]], add=True)
            pltpu.async_copy(g_hbm.at[colg.at[pn, b]], bufs[b], gsems[b])

        @pl.when(g + 2 < NGRP)
        def _():
            pltpu.async_copy(col_hbm.at[wid, g + 2], colg.at[p], isem)
            pltpu.async_copy(row_hbm.at[wid, g + 2], rowg.at[p], isem)

        return carry

    lax.fori_loop(0, NGRP - 1, body, 0)

    pl_ = lax.rem(NGRP - 1, 2)
    for b in range(NBUF):
        pltpu.make_async_copy(g_hbm.at[colg.at[pl_, b]], bufs[b],
                              gsems[b]).wait()
        pltpu.sync_copy(bufs[b], accum.at[rowg.at[pl_, b]], add=True)

    plsc.subcore_barrier()
    pltpu.sync_copy(accum.at[pl.ds(r0, ROWS_PER)],
                    out_hbm.at[cid, pl.ds(r0, ROWS_PER)])


_B = 2000
_GRID = N // _B


def _tc_prep_body(x_ref, d16_ref, dinvb_ref, g0_ref):
    deg = d16_ref[0, :, 0:1] + d16_ref[1, :, 0:1]
    dinv = jnp.where(deg > 0.0, lax.rsqrt(jnp.maximum(deg, 1e-12)), 0.0)
    dinvb = jnp.broadcast_to(dinv, (_B, C))
    dinvb_ref[...] = dinvb
    g0_ref[...] = dinvb * x_ref[...]


def _tc_combine_body(first, last, yp_ref, dinvb_ref, txm2_ref, *refs):
    y = yp_ref[0] + yp_ref[1]
    dinvb = dinvb_ref[...]
    if first:
        tx = -(dinvb * y)
    else:
        tx = -2.0 * (dinvb * y) - txm2_ref[...]
    refs[0][...] = tx
    if not last:
        refs[1][...] = dinvb * tx


def _tc_final_body(yp_ref, dinvb_ref, tx0_ref, tx1_ref, tx2_ref, tx3_ref,
                   tx4_ref, w_ref, b_ref, o_ref):
    tx5 = -2.0 * (dinvb_ref[...] * (yp_ref[0] + yp_ref[1])) - tx3_ref[...]
    txs = (tx0_ref[...], tx1_ref[...], tx2_ref[...], tx3_ref[...],
           tx4_ref[...], tx5)
    h = b_ref[...]
    for k in range(K):
        h = h + jnp.dot(txs[k], w_ref[k], preferred_element_type=jnp.float32)
    m = jnp.mean(h, axis=1, keepdims=True)
    cen = h - m
    v = jnp.mean(cen * cen, axis=1, keepdims=True)
    o_ref[...] = jnp.maximum(cen * lax.rsqrt(v + EPS), 0.0)


_row_spec = pl.BlockSpec((_B, C), lambda i: (i, 0))
_yp_spec = pl.BlockSpec((NC, _B, C), lambda i: (0, i, 0))

_tc_prep = pl.pallas_call(
    _tc_prep_body,
    grid=(_GRID,),
    in_specs=[_row_spec, _yp_spec],
    out_specs=[_row_spec, _row_spec],
    out_shape=[jax.ShapeDtypeStruct((N, C), jnp.float32)] * 2,
)

_BC = 5000
_rowc_spec = pl.BlockSpec((_BC, C), lambda i: (i, 0))
_ypc_spec = pl.BlockSpec((NC, _BC, C), lambda i: (0, i, 0))

_tc_combine_first = pl.pallas_call(
    functools.partial(_tc_combine_body, True, False),
    grid=(N // _BC,),
    in_specs=[_ypc_spec, _rowc_spec, _rowc_spec],
    out_specs=[_rowc_spec, _rowc_spec],
    out_shape=[jax.ShapeDtypeStruct((N, C), jnp.float32)] * 2,
)

_tc_combine_rest = pl.pallas_call(
    functools.partial(_tc_combine_body, False, False),
    grid=(N // _BC,),
    in_specs=[_ypc_spec, _rowc_spec, _rowc_spec],
    out_specs=[_rowc_spec, _rowc_spec],
    out_shape=[jax.ShapeDtypeStruct((N, C), jnp.float32)] * 2,
)

_tc_final = pl.pallas_call(
    _tc_final_body,
    grid=(_GRID,),
    in_specs=[_yp_spec] + [_row_spec] * K
    + [pl.BlockSpec((K, C, C), lambda i: (0, 0, 0)),
       pl.BlockSpec((1, C), lambda i: (0, 0))],
    out_specs=_row_spec,
    out_shape=jax.ShapeDtypeStruct((N, C), jnp.float32),
)


def kernel(x, adj_indices, W, b):
    pad = NGRP * NBUF * CH - EPW
    pad_rows = jnp.broadcast_to(N + (jnp.arange(pad) % (NPAD - N)), (NW, pad))
    pad_cols = jnp.broadcast_to((jnp.arange(pad) * 97) % N, (NW, pad))
    row = jnp.concatenate(
        [adj_indices[0].reshape(NW, EPW), pad_rows.astype(jnp.int32)], axis=1)
    row = row.reshape(NW, NGRP, NBUF, CH)
    col = jnp.concatenate(
        [adj_indices[1].reshape(NW, EPW), pad_cols.astype(jnp.int32)], axis=1)
    col = col.reshape(NW, NGRP, NBUF, CH)
    row_d = adj_indices[0].reshape(NW, NGRP_D, NBUF_D, CH_D)

    d16 = _sc_degree(row_d)
    dinvb, g = _tc_prep(x, d16)

    txs = [x]
    for k in range(1, K - 1):
        yp = _sc_segsum(g, col, row)
        if k == 1:
            tx, g = _tc_combine_first(yp, dinvb, txs[0])
        else:
            tx, g = _tc_combine_rest(yp, dinvb, txs[k - 2])
        txs.append(tx)

    yp = _sc_segsum(g, col, row)
    return _tc_final(yp, dinvb, *txs, W, b.reshape(1, C))
